# baseline (speedup 1.0000x reference)
import dataclasses
import functools

import jax
import jax.numpy as jnp
from jax import lax
from jax.experimental import pallas as pl
from jax.experimental.pallas import tpu as pltpu
from jax.experimental.pallas import tpu_sc as plsc

N = 10000
E = 320000
D = 128
H = 8
U = 16
HP = 16
DW = D + HP
CHUNK = 64
NC = 2
NS = 16
NW = NC * NS
CPT = 160
EPT = CPT * CHUNK
E_PAD = NW * EPT
NCH = E_PAD // CHUNK
CPH = 16
NPH = CPT // CPH
NP = 10240
RPS = NP // NS

_SC_CP = pltpu.CompilerParams()
if "needs_layout_passes" in pltpu.CompilerParams.__dataclass_fields__:
    _SC_CP = dataclasses.replace(_SC_CP, needs_layout_passes=False)
if "use_tc_tiling_on_sc" in pltpu.CompilerParams.__dataclass_fields__:
    _SC_CP = dataclasses.replace(_SC_CP, use_tc_tiling_on_sc=False)


def _tc_project(xp, w):
    rb = 1024

    def body(x_ref, w_ref, xn_ref, xw_ref):
        xn = jnp.dot(
            x_ref[...], w_ref[...], preferred_element_type=jnp.float32)
        xn_ref[...] = xn
        xw_ref[...] = jnp.concatenate(
            [xn, jnp.zeros((rb, DW - D), jnp.float32)], axis=1)

    return pl.pallas_call(
        body,
        grid=(NP // rb,),
        in_specs=[
            pl.BlockSpec((rb, D), lambda i: (i, 0)),
            pl.BlockSpec((D, H * U), lambda i: (0, 0)),
        ],
        out_specs=[
            pl.BlockSpec((rb, D), lambda i: (i, 0)),
            pl.BlockSpec((rb, DW), lambda i: (i, 0)),
        ],
        out_shape=[
            jax.ShapeDtypeStruct((NP, D), jnp.float32),
            jax.ShapeDtypeStruct((NP, DW), jnp.float32),
        ],
    )(xp, w)


def _sc_fused(xn, xnw, tgt2d, src2d, ka1, tba, zerosw):
    mesh = plsc.VectorSubcoreMesh(core_axis_name="c", subcore_axis_name="s")

    @functools.partial(
        pl.kernel,
        out_type=jax.ShapeDtypeStruct((NC, NP, DW), jnp.float32),
        mesh=mesh,
        compiler_params=_SC_CP,
        scratch_types=[
            pltpu.VMEM_SHARED((NP, DW), jnp.float32),
            pltpu.VMEM((CPH, CHUNK), jnp.int32),
            pltpu.VMEM((CPH, CHUNK), jnp.int32),
            pltpu.VMEM((CHUNK, D), jnp.float32),
            pltpu.VMEM((CHUNK, D), jnp.float32),
            pltpu.VMEM((CHUNK, DW), jnp.float32),
            pltpu.VMEM((CHUNK, DW), jnp.float32),

            pltpu.VMEM((D,), jnp.float32),
            pltpu.VMEM((D,), jnp.float32),
            pltpu.SemaphoreType.DMA,
            pltpu.SemaphoreType.DMA,
            pltpu.SemaphoreType.DMA,
            pltpu.SemaphoreType.DMA,
        ],
    )
    def k(xn_hbm, xnw_hbm, tgt_hbm, src_hbm, ka1_hbm, tba_hbm, z_hbm,
          out_hbm,
          acc, tgtv, srcv, ft0, ft1, fs0, fs1, ka1_v, tba_v,
          sf0, sf1, ss0, ss1):
        c = lax.axis_index("c")
        s = lax.axis_index("s")
        w = s * NC + c
        ft = (ft0, ft1)
        fs = (fs0, fs1)
        sf = (sf0, sf1)
        ss = (ss0, ss1)
        pltpu.sync_copy(ka1_hbm, ka1_v)
        pltpu.sync_copy(tba_hbm, tba_v)
        ka = [ka1_v[pl.ds(h * U, U)] for h in range(H)]
        tb = [tba_v[pl.ds(h * U, U)] for h in range(H)]
        pltpu.sync_copy(
            z_hbm.at[pl.ds(s * RPS, RPS)], acc.at[pl.ds(s * RPS, RPS)])
        plsc.subcore_barrier()

        for ph in range(NPH):
            pltpu.sync_copy(
                tgt_hbm.at[pl.ds(w * CPT + ph * CPH, CPH)], tgtv)
            pltpu.sync_copy(
                src_hbm.at[pl.ds(w * CPT + ph * CPH, CPH)], srcv)

            def descs(r, b):
                return (
                    pltpu.make_async_copy(xn_hbm.at[tgtv.at[r]], ft[b], sf[b]),
                    pltpu.make_async_copy(
                        xnw_hbm.at[srcv.at[r]], fs[b], ss[b]))

            def issue(r, b):
                d1, d2 = descs(r, b)
                d1.start()
                d2.start()

            def compute_tail(r, b):
                d1, d2 = descs(r, b)
                d1.wait()
                d2.wait()

                @plsc.parallel_loop(0, CHUNK, unroll=2)
                def _(i):
                    lane = lax.iota(jnp.int32, HP)
                    row = jnp.zeros((HP,), jnp.float32)
                    for h in range(H):
                        z = (ft[b][i, pl.ds(h * U, U)]
                             + fs[b][i, pl.ds(h * U, U)]) + tb[h]
                        t = jnp.maximum(z, 0.2 * z) * ka[h]
                        row = jnp.where(lane == h, jnp.sum(t), row)
                    p16 = jnp.where(lane < H, jnp.exp(row), 0.0)
                    fs[b][i, pl.ds(D, HP)] = p16
                    for h in range(H):
                        fs[b][i, pl.ds(h * U, U)] = (
                            fs[b][i, pl.ds(h * U, U)] * p16[h])

                pltpu.sync_copy(fs[b], acc.at[tgtv.at[r]], add=True)

            issue(0, 0)

            @pl.loop(0, CPH, step=2)
            def _(rr):
                issue(rr + 1, 1)
                compute_tail(rr, 0)

                @pl.when(rr + 2 < CPH)
                def _():
                    issue(rr + 2, 0)

                compute_tail(rr + 1, 1)

        plsc.subcore_barrier()
        pltpu.sync_copy(
            acc.at[pl.ds(s * RPS, RPS)],
            out_hbm.at[c, pl.ds(s * RPS, RPS)])

    return k(xn, xnw, tgt2d, src2d, ka1, tba, zerosw)


def _tc_finish(accw, bias):
    rb = 1000

    def body(a_ref, b_ref, o_ref):
        aw = a_ref[0] + a_ref[1]
        msg = aw[:, :D]
        ssum = aw[:, D:D + H]
        rinvx = jnp.repeat(1.0 / (ssum + 1e-7), U, axis=1)
        o_ref[...] = jax.nn.gelu(msg * rinvx + b_ref[...])

    return pl.pallas_call(
        body,
        grid=(N // rb,),
        in_specs=[
            pl.BlockSpec((NC, rb, DW), lambda i: (0, i, 0)),
            pl.BlockSpec((1, D), lambda i: (0, 0)),
        ],
        out_specs=pl.BlockSpec((rb, D), lambda i: (i, 0)),
        out_shape=jax.ShapeDtypeStruct((N, D), jnp.float32),
    )(accw, bias.reshape(1, D))


def kernel(x, edges, kernel, kernel_attention1, bias_attention, bias):
    w = kernel.reshape(D, H * U)
    ka1 = kernel_attention1.reshape(H * U)
    tba = 2.0 * bias_attention.reshape(H * U)
    pad = N + (jnp.arange(E_PAD - E, dtype=jnp.int32) % (NP - N))
    tgt2d = jnp.concatenate([edges[:, 1], pad]).reshape(NCH, CHUNK)
    src2d = jnp.concatenate([edges[:, 0], pad]).reshape(NCH, CHUNK)
    xp = jnp.pad(x, ((0, NP - N), (0, 0)))
    zerosw = jnp.zeros((NP, DW), jnp.float32)

    xn, xnw = _tc_project(xp, w)
    accw = _sc_fused(xn, xnw, tgt2d, src2d, ka1, tba, zerosw)
    return _tc_finish(accw, bias)


# speedup vs baseline: 1.2096x; 1.2096x over previous
import dataclasses
import functools

import jax
import jax.numpy as jnp
from jax import lax
from jax.experimental import pallas as pl
from jax.experimental.pallas import tpu as pltpu
from jax.experimental.pallas import tpu_sc as plsc

N = 10000
E = 320000
D = 128
H = 8
U = 16
HP = 16
CHUNK = 64
NC = 2
NS = 16
NW = NC * NS
CPT = 160
EPT = CPT * CHUNK
E_PAD = NW * EPT
NCH = E_PAD // CHUNK
CPH = 32
NPH = CPT // CPH
NP = 10112
RPS = NP // NS

_SC_CP = pltpu.CompilerParams()
if "needs_layout_passes" in pltpu.CompilerParams.__dataclass_fields__:
    _SC_CP = dataclasses.replace(_SC_CP, needs_layout_passes=False)
if "use_tc_tiling_on_sc" in pltpu.CompilerParams.__dataclass_fields__:
    _SC_CP = dataclasses.replace(_SC_CP, use_tc_tiling_on_sc=False)


def _tc_project(xp, w):
    rb = 1264

    def body(x_ref, w_ref, xn_ref):
        xn_ref[...] = jnp.dot(
            x_ref[...], w_ref[...], preferred_element_type=jnp.float32)

    return pl.pallas_call(
        body,
        grid=(NP // rb,),
        in_specs=[
            pl.BlockSpec((rb, D), lambda i: (i, 0)),
            pl.BlockSpec((D, H * U), lambda i: (0, 0)),
        ],
        out_specs=pl.BlockSpec((rb, H * U), lambda i: (i, 0)),
        out_shape=jax.ShapeDtypeStruct((NP, H * U), jnp.float32),
    )(xp, w)


def _sc_fused(xn, tgt2d, src2d, ka1, tba, zeros16, zeros128):
    mesh = plsc.VectorSubcoreMesh(core_axis_name="c", subcore_axis_name="s")

    @functools.partial(
        pl.kernel,
        out_type=(
            jax.ShapeDtypeStruct((NC, NP, HP), jnp.float32),
            jax.ShapeDtypeStruct((NC, NP, D), jnp.float32),
        ),
        mesh=mesh,
        compiler_params=_SC_CP,
        scratch_types=[
            pltpu.VMEM_SHARED((NP, HP), jnp.float32),
            pltpu.VMEM_SHARED((NP, D), jnp.float32),
            pltpu.VMEM((CPH, CHUNK), jnp.int32),
            pltpu.VMEM((CPH, CHUNK), jnp.int32),
            pltpu.VMEM((CHUNK, D), jnp.float32),
            pltpu.VMEM((CHUNK, D), jnp.float32),
            pltpu.VMEM((CHUNK, D), jnp.float32),
            pltpu.VMEM((CHUNK, D), jnp.float32),
            pltpu.VMEM((CHUNK, HP), jnp.float32),
            pltpu.VMEM((CHUNK, HP), jnp.float32),
            pltpu.VMEM((D,), jnp.float32),
            pltpu.VMEM((D,), jnp.float32),
            pltpu.SemaphoreType.DMA,
            pltpu.SemaphoreType.DMA,
            pltpu.SemaphoreType.DMA,
            pltpu.SemaphoreType.DMA,
            pltpu.SemaphoreType.DMA,
            pltpu.SemaphoreType.DMA,
        ],
    )
    def k(xn_hbm, tgt_hbm, src_hbm, ka1_hbm, tba_hbm, z16_hbm, z128_hbm,
          ssum_hbm, out_hbm,
          acc16, acc128, tgtv, srcv, ft0, ft1, fs0, fs1, p0, p1, ka1_v, tba_v,
          sf0, sf1, ss0, ss1, sp0, sp1):
        c = lax.axis_index("c")
        s = lax.axis_index("s")
        w = s * NC + c
        ft = (ft0, ft1)
        fs = (fs0, fs1)
        pv = (p0, p1)
        sf = (sf0, sf1)
        ss = (ss0, ss1)
        sp = (sp0, sp1)
        pltpu.sync_copy(ka1_hbm, ka1_v)
        pltpu.sync_copy(tba_hbm, tba_v)
        ka = [ka1_v[pl.ds(h * U, U)] for h in range(H)]
        tb = [tba_v[pl.ds(h * U, U)] for h in range(H)]
        pltpu.sync_copy(
            z16_hbm.at[pl.ds(s * RPS, RPS)], acc16.at[pl.ds(s * RPS, RPS)])
        pltpu.sync_copy(
            z128_hbm.at[pl.ds(s * RPS, RPS)], acc128.at[pl.ds(s * RPS, RPS)])
        plsc.subcore_barrier()

        for ph in range(NPH):
            pltpu.sync_copy(
                tgt_hbm.at[pl.ds(w * CPT + ph * CPH, CPH)], tgtv)
            pltpu.sync_copy(
                src_hbm.at[pl.ds(w * CPT + ph * CPH, CPH)], srcv)

            def descs(r, b):
                return (
                    pltpu.make_async_copy(xn_hbm.at[tgtv.at[r]], ft[b], sf[b]),
                    pltpu.make_async_copy(xn_hbm.at[srcv.at[r]], fs[b], ss[b]))

            def issue(r, b):
                d1, d2 = descs(r, b)
                d1.start()
                d2.start()

            def pdesc(r, b):
                return pltpu.make_async_copy(
                    pv[b], acc16.at[tgtv.at[r]], sp[b])

            def compute_tail(r, b):
                d1, d2 = descs(r, b)
                d1.wait()
                d2.wait()

                @pl.when(r >= 2)
                def _():
                    pdesc(r, b).wait()

                @plsc.parallel_loop(0, CHUNK, unroll=2)
                def _(i):
                    lane = lax.iota(jnp.int32, HP)
                    row = jnp.zeros((HP,), jnp.float32)
                    for h in range(H):
                        z = (ft[b][i, pl.ds(h * U, U)]
                             + fs[b][i, pl.ds(h * U, U)]) + tb[h]
                        t = jnp.maximum(z, 0.2 * z) * ka[h]
                        row = jnp.where(lane == h, jnp.sum(t), row)
                    p16 = jnp.where(lane < H, jnp.exp(row), 0.0)
                    pv[b][i, :] = p16
                    for h in range(H):
                        fs[b][i, pl.ds(h * U, U)] = (
                            fs[b][i, pl.ds(h * U, U)] * p16[h])

                pdesc(r, b).start()
                pltpu.sync_copy(fs[b], acc128.at[tgtv.at[r]], add=True)

            issue(0, 0)

            @pl.loop(0, CPH, step=2)
            def _(rr):
                issue(rr + 1, 1)
                compute_tail(rr, 0)

                @pl.when(rr + 2 < CPH)
                def _():
                    issue(rr + 2, 0)

                compute_tail(rr + 1, 1)

            pdesc(CPH - 2, 0).wait()
            pdesc(CPH - 1, 1).wait()

        plsc.subcore_barrier()
        pltpu.sync_copy(
            acc16.at[pl.ds(s * RPS, RPS)],
            ssum_hbm.at[c, pl.ds(s * RPS, RPS)])
        pltpu.sync_copy(
            acc128.at[pl.ds(s * RPS, RPS)],
            out_hbm.at[c, pl.ds(s * RPS, RPS)])

    return k(xn, tgt2d, src2d, ka1, tba, zeros16, zeros128)


def _tc_finish(ssum_p, acc, bias):
    rb = 1000

    def body(s_ref, a_ref, b_ref, o_ref):
        rinv = 1.0 / (s_ref[0, :, :H] + s_ref[1, :, :H] + 1e-7)
        rinvx = jnp.repeat(rinv, U, axis=1)
        o_ref[...] = jax.nn.gelu(
            (a_ref[0] + a_ref[1]) * rinvx + b_ref[...])

    return pl.pallas_call(
        body,
        grid=(N // rb,),
        in_specs=[
            pl.BlockSpec((NC, rb, HP), lambda i: (0, i, 0)),
            pl.BlockSpec((NC, rb, D), lambda i: (0, i, 0)),
            pl.BlockSpec((1, D), lambda i: (0, 0)),
        ],
        out_specs=pl.BlockSpec((rb, D), lambda i: (i, 0)),
        out_shape=jax.ShapeDtypeStruct((N, D), jnp.float32),
    )(ssum_p, acc, bias.reshape(1, D))


def kernel(x, edges, kernel, kernel_attention1, bias_attention, bias):
    w = kernel.reshape(D, H * U)
    ka1 = kernel_attention1.reshape(H * U)
    tba = 2.0 * bias_attention.reshape(H * U)
    pad = N + (jnp.arange(E_PAD - E, dtype=jnp.int32) % (NP - N))
    tgt2d = jnp.concatenate([edges[:, 1], pad]).reshape(NCH, CHUNK)
    src2d = jnp.concatenate([edges[:, 0], pad]).reshape(NCH, CHUNK)
    xp = jnp.pad(x, ((0, NP - N), (0, 0)))
    zeros16 = jnp.zeros((NP, HP), jnp.float32)
    zeros128 = jnp.zeros((NP, D), jnp.float32)

    xn = _tc_project(xp, w)
    ssum_p, acc = _sc_fused(xn, tgt2d, src2d, ka1, tba, zeros16, zeros128)
    return _tc_finish(ssum_p, acc, bias)


# speedup vs baseline: 1.2098x; 1.0002x over previous
import dataclasses
import functools

import jax
import jax.numpy as jnp
from jax import lax
from jax.experimental import pallas as pl
from jax.experimental.pallas import tpu as pltpu
from jax.experimental.pallas import tpu_sc as plsc

N = 10000
E = 320000
D = 128
H = 8
U = 16
HP = 16
CHUNK = 64
NC = 2
NS = 16
NW = NC * NS
CPT = 160
EPT = CPT * CHUNK
E_PAD = NW * EPT
NCH = E_PAD // CHUNK
CPH = 32
NPH = CPT // CPH
NP = 10112
RPS = NP // NS

_SC_CP = pltpu.CompilerParams()
if "needs_layout_passes" in pltpu.CompilerParams.__dataclass_fields__:
    _SC_CP = dataclasses.replace(_SC_CP, needs_layout_passes=False)
if "use_tc_tiling_on_sc" in pltpu.CompilerParams.__dataclass_fields__:
    _SC_CP = dataclasses.replace(_SC_CP, use_tc_tiling_on_sc=False)


def _tc_project(xp, w):
    rb = 1264

    def body(x_ref, w_ref, xn_ref):
        xn_ref[...] = jnp.dot(
            x_ref[...], w_ref[...], preferred_element_type=jnp.float32)

    return pl.pallas_call(
        body,
        grid=(NP // rb,),
        in_specs=[
            pl.BlockSpec((rb, D), lambda i: (i, 0)),
            pl.BlockSpec((D, H * U), lambda i: (0, 0)),
        ],
        out_specs=pl.BlockSpec((rb, H * U), lambda i: (i, 0)),
        out_shape=jax.ShapeDtypeStruct((NP, H * U), jnp.float32),
    )(xp, w)


def _sc_fused(xn, tgt2d, src2d, ka1, tba, zeros16, zeros128):
    mesh = plsc.VectorSubcoreMesh(core_axis_name="c", subcore_axis_name="s")

    @functools.partial(
        pl.kernel,
        out_type=(
            jax.ShapeDtypeStruct((NC, NP, HP), jnp.float32),
            jax.ShapeDtypeStruct((NC, NP, D), jnp.float32),
        ),
        mesh=mesh,
        compiler_params=_SC_CP,
        scratch_types=[
            pltpu.VMEM_SHARED((NP, HP), jnp.float32),
            pltpu.VMEM_SHARED((NP, D), jnp.float32),
            pltpu.VMEM((CPH, CHUNK), jnp.int32),
            pltpu.VMEM((CPH, CHUNK), jnp.int32),
            pltpu.VMEM((CHUNK, D), jnp.float32),
            pltpu.VMEM((CHUNK, D), jnp.float32),
            pltpu.VMEM((CHUNK, D), jnp.float32),
            pltpu.VMEM((CHUNK, D), jnp.float32),
            pltpu.VMEM((CHUNK, HP), jnp.float32),
            pltpu.VMEM((CHUNK, HP), jnp.float32),
            pltpu.VMEM((D,), jnp.float32),
            pltpu.VMEM((D,), jnp.float32),
            pltpu.SemaphoreType.DMA,
            pltpu.SemaphoreType.DMA,
            pltpu.SemaphoreType.DMA,
            pltpu.SemaphoreType.DMA,
            pltpu.SemaphoreType.DMA,
            pltpu.SemaphoreType.DMA,
        ],
    )
    def k(xn_hbm, tgt_hbm, src_hbm, ka1_hbm, tba_hbm, z16_hbm, z128_hbm,
          ssum_hbm, out_hbm,
          acc16, acc128, tgtv, srcv, ft0, ft1, fs0, fs1, p0, p1, ka1_v, tba_v,
          sf0, sf1, ss0, ss1, sp0, sp1):
        c = lax.axis_index("c")
        s = lax.axis_index("s")
        w = s * NC + c
        ft = (ft0, ft1)
        fs = (fs0, fs1)
        pv = (p0, p1)
        sf = (sf0, sf1)
        ss = (ss0, ss1)
        sp = (sp0, sp1)
        pltpu.sync_copy(ka1_hbm, ka1_v)
        pltpu.sync_copy(tba_hbm, tba_v)
        ka = [ka1_v[pl.ds(h * U, U)] for h in range(H)]
        tb = [tba_v[pl.ds(h * U, U)] for h in range(H)]
        pltpu.sync_copy(
            z16_hbm.at[pl.ds(s * RPS, RPS)], acc16.at[pl.ds(s * RPS, RPS)])
        pltpu.sync_copy(
            z128_hbm.at[pl.ds(s * RPS, RPS)], acc128.at[pl.ds(s * RPS, RPS)])
        plsc.subcore_barrier()

        for ph in range(NPH):
            pltpu.sync_copy(
                tgt_hbm.at[pl.ds(w * CPT + ph * CPH, CPH)], tgtv)
            pltpu.sync_copy(
                src_hbm.at[pl.ds(w * CPT + ph * CPH, CPH)], srcv)

            def descs(r, b):
                return (
                    pltpu.make_async_copy(xn_hbm.at[tgtv.at[r]], ft[b], sf[b]),
                    pltpu.make_async_copy(xn_hbm.at[srcv.at[r]], fs[b], ss[b]))

            def issue(r, b):
                d1, d2 = descs(r, b)
                d1.start()
                d2.start()

            def pdesc(r, b):
                return pltpu.make_async_copy(
                    pv[b], acc16.at[tgtv.at[r]], sp[b])

            def compute_tail(r, b):
                d1, d2 = descs(r, b)
                d1.wait()
                d2.wait()

                @pl.when(r >= 2)
                def _():
                    pdesc(r, b).wait()

                @plsc.parallel_loop(0, CHUNK, unroll=2)
                def _(i):
                    lane = lax.iota(jnp.int32, HP)
                    row = jnp.zeros((HP,), jnp.float32)
                    for h in range(H):
                        z = (ft[b][i, pl.ds(h * U, U)]
                             + fs[b][i, pl.ds(h * U, U)]) + tb[h]
                        t = jnp.maximum(z, 0.2 * z) * ka[h]
                        row = jnp.where(lane == h, jnp.sum(t), row)
                    p16 = jnp.where(lane < H, jnp.exp(row), 0.0)
                    pv[b][i, :] = p16
                    for h in range(H):
                        fs[b][i, pl.ds(h * U, U)] = (
                            fs[b][i, pl.ds(h * U, U)] * p16[h])

                pltpu.async_copy(
                    pv[b], acc16.at[tgtv.at[r]], sp[b], add=True)
                pltpu.sync_copy(fs[b], acc128.at[tgtv.at[r]], add=True)

            issue(0, 0)

            @pl.loop(0, CPH, step=2)
            def _(rr):
                issue(rr + 1, 1)
                compute_tail(rr, 0)

                @pl.when(rr + 2 < CPH)
                def _():
                    issue(rr + 2, 0)

                compute_tail(rr + 1, 1)

            pdesc(CPH - 2, 0).wait()
            pdesc(CPH - 1, 1).wait()

        plsc.subcore_barrier()
        pltpu.sync_copy(
            acc16.at[pl.ds(s * RPS, RPS)],
            ssum_hbm.at[c, pl.ds(s * RPS, RPS)])
        pltpu.sync_copy(
            acc128.at[pl.ds(s * RPS, RPS)],
            out_hbm.at[c, pl.ds(s * RPS, RPS)])

    return k(xn, tgt2d, src2d, ka1, tba, zeros16, zeros128)


def _tc_finish(ssum_p, acc, bias):
    rb = 1000

    def body(s_ref, a_ref, b_ref, o_ref):
        rinv = 1.0 / (s_ref[0, :, :H] + s_ref[1, :, :H] + 1e-7)
        rinvx = jnp.repeat(rinv, U, axis=1)
        o_ref[...] = jax.nn.gelu(
            (a_ref[0] + a_ref[1]) * rinvx + b_ref[...])

    return pl.pallas_call(
        body,
        grid=(N // rb,),
        in_specs=[
            pl.BlockSpec((NC, rb, HP), lambda i: (0, i, 0)),
            pl.BlockSpec((NC, rb, D), lambda i: (0, i, 0)),
            pl.BlockSpec((1, D), lambda i: (0, 0)),
        ],
        out_specs=pl.BlockSpec((rb, D), lambda i: (i, 0)),
        out_shape=jax.ShapeDtypeStruct((N, D), jnp.float32),
    )(ssum_p, acc, bias.reshape(1, D))


def kernel(x, edges, kernel, kernel_attention1, bias_attention, bias):
    w = kernel.reshape(D, H * U)
    ka1 = kernel_attention1.reshape(H * U)
    tba = 2.0 * bias_attention.reshape(H * U)
    pad = N + (jnp.arange(E_PAD - E, dtype=jnp.int32) % (NP - N))
    tgt2d = jnp.concatenate([edges[:, 1], pad]).reshape(NCH, CHUNK)
    src2d = jnp.concatenate([edges[:, 0], pad]).reshape(NCH, CHUNK)
    xp = jnp.pad(x, ((0, NP - N), (0, 0)))
    zeros16 = jnp.zeros((NP, HP), jnp.float32)
    zeros128 = jnp.zeros((NP, D), jnp.float32)

    xn = _tc_project(xp, w)
    ssum_p, acc = _sc_fused(xn, tgt2d, src2d, ka1, tba, zeros16, zeros128)
    return _tc_finish(ssum_p, acc, bias)


# speedup vs baseline: 1.2756x; 1.0544x over previous
import dataclasses
import functools

import jax
import jax.numpy as jnp
from jax import lax
from jax.experimental import pallas as pl
from jax.experimental.pallas import tpu as pltpu
from jax.experimental.pallas import tpu_sc as plsc

N = 10000
E = 320000
D = 128
H = 8
U = 16
HP = 16
CHUNK = 40
NC = 2
NS = 16
NW = NC * NS
EPT = 10240
CPT = EPT // CHUNK
E_PAD = NW * EPT
NCH = E_PAD // CHUNK
CPH = 64
NPH = CPT // CPH
NP = 10112
RPS = NP // NS

_SC_CP = pltpu.CompilerParams()
if "needs_layout_passes" in pltpu.CompilerParams.__dataclass_fields__:
    _SC_CP = dataclasses.replace(_SC_CP, needs_layout_passes=False)
if "use_tc_tiling_on_sc" in pltpu.CompilerParams.__dataclass_fields__:
    _SC_CP = dataclasses.replace(_SC_CP, use_tc_tiling_on_sc=False)


def _tc_project(xp, w):
    rb = 1264

    def body(x_ref, w_ref, xn_ref):
        xn_ref[...] = jnp.dot(
            x_ref[...], w_ref[...], preferred_element_type=jnp.float32)

    return pl.pallas_call(
        body,
        grid=(NP // rb,),
        in_specs=[
            pl.BlockSpec((rb, D), lambda i: (i, 0)),
            pl.BlockSpec((D, H * U), lambda i: (0, 0)),
        ],
        out_specs=pl.BlockSpec((rb, H * U), lambda i: (i, 0)),
        out_shape=jax.ShapeDtypeStruct((NP, H * U), jnp.float32),
    )(xp, w)


def _sc_fused(xn, tgt2d, src2d, ka1, tba, zeros16, zeros128):
    mesh = plsc.VectorSubcoreMesh(core_axis_name="c", subcore_axis_name="s")

    @functools.partial(
        pl.kernel,
        out_type=(
            jax.ShapeDtypeStruct((NC, NP, HP), jnp.float32),
            jax.ShapeDtypeStruct((NC, NP, D), jnp.float32),
        ),
        mesh=mesh,
        compiler_params=_SC_CP,
        scratch_types=[
            pltpu.VMEM_SHARED((NP, HP), jnp.float32),
            pltpu.VMEM_SHARED((NP, D), jnp.float32),
            pltpu.VMEM((CPH, CHUNK), jnp.int32),
            pltpu.VMEM((CPH, CHUNK), jnp.int32),
            pltpu.VMEM((CHUNK, D), jnp.float32),
            pltpu.VMEM((CHUNK, D), jnp.float32),
            pltpu.VMEM((CHUNK, D), jnp.float32),
            pltpu.VMEM((CHUNK, D), jnp.float32),
            pltpu.VMEM((CHUNK, D), jnp.float32),
            pltpu.VMEM((CHUNK, D), jnp.float32),
            pltpu.VMEM((CHUNK, HP), jnp.float32),
            pltpu.VMEM((CHUNK, HP), jnp.float32),
            pltpu.VMEM((D,), jnp.float32),
            pltpu.VMEM((D,), jnp.float32),
            pltpu.SemaphoreType.DMA,
            pltpu.SemaphoreType.DMA,
            pltpu.SemaphoreType.DMA,
            pltpu.SemaphoreType.DMA,
            pltpu.SemaphoreType.DMA,
            pltpu.SemaphoreType.DMA,
            pltpu.SemaphoreType.DMA,
            pltpu.SemaphoreType.DMA,
        ],
    )
    def k(xn_hbm, tgt_hbm, src_hbm, ka1_hbm, tba_hbm, z16_hbm, z128_hbm,
          ssum_hbm, out_hbm,
          acc16, acc128, tgtv, srcv, ft0, ft1, fs0, fs1, mg0, mg1, p0, p1,
          ka1_v, tba_v,
          sf0, sf1, ss0, ss1, sp0, sp1, sm0, sm1):
        c = lax.axis_index("c")
        s = lax.axis_index("s")
        w = s * NC + c
        ft = (ft0, ft1)
        fs = (fs0, fs1)
        mg = (mg0, mg1)
        pv = (p0, p1)
        sf = (sf0, sf1)
        ss = (ss0, ss1)
        sp = (sp0, sp1)
        sm = (sm0, sm1)
        pltpu.sync_copy(ka1_hbm, ka1_v)
        pltpu.sync_copy(tba_hbm, tba_v)
        ka = [ka1_v[pl.ds(h * U, U)] for h in range(H)]
        tb = [tba_v[pl.ds(h * U, U)] for h in range(H)]
        pltpu.sync_copy(
            z16_hbm.at[pl.ds(s * RPS, RPS)], acc16.at[pl.ds(s * RPS, RPS)])
        pltpu.sync_copy(
            z128_hbm.at[pl.ds(s * RPS, RPS)], acc128.at[pl.ds(s * RPS, RPS)])
        plsc.subcore_barrier()

        for ph in range(NPH):
            pltpu.sync_copy(
                tgt_hbm.at[pl.ds(w * CPT + ph * CPH, CPH)], tgtv)
            pltpu.sync_copy(
                src_hbm.at[pl.ds(w * CPT + ph * CPH, CPH)], srcv)

            def descs(r, b):
                return (
                    pltpu.make_async_copy(xn_hbm.at[tgtv.at[r]], ft[b], sf[b]),
                    pltpu.make_async_copy(xn_hbm.at[srcv.at[r]], fs[b], ss[b]))

            def issue(r, b):
                d1, d2 = descs(r, b)
                d1.start()
                d2.start()

            def pdesc(r, b):
                return pltpu.make_async_copy(
                    pv[b], acc16.at[tgtv.at[r]], sp[b])

            def mdesc(r, b):
                return pltpu.make_async_copy(
                    mg[b], acc128.at[tgtv.at[r]], sm[b])

            def compute_tail(r, b):
                d1, d2 = descs(r, b)
                d1.wait()
                d2.wait()

                @pl.when(r >= 2)
                def _():
                    pdesc(r, b).wait()
                    mdesc(r, b).wait()

                @plsc.parallel_loop(0, CHUNK, unroll=2)
                def _(i):
                    lane = lax.iota(jnp.int32, HP)
                    row = jnp.zeros((HP,), jnp.float32)
                    for h in range(H):
                        z = (ft[b][i, pl.ds(h * U, U)]
                             + fs[b][i, pl.ds(h * U, U)]) + tb[h]
                        t = jnp.maximum(z, 0.2 * z) * ka[h]
                        row = jnp.where(lane == h, jnp.sum(t), row)
                    p16 = jnp.where(lane < H, jnp.exp(row), 0.0)
                    pv[b][i, :] = p16
                    for h in range(H):
                        mg[b][i, pl.ds(h * U, U)] = (
                            fs[b][i, pl.ds(h * U, U)] * p16[h])

                pltpu.async_copy(
                    pv[b], acc16.at[tgtv.at[r]], sp[b], add=True)
                pltpu.async_copy(
                    mg[b], acc128.at[tgtv.at[r]], sm[b], add=True)

            issue(0, 0)

            @pl.loop(0, CPH, step=2)
            def _(rr):
                issue(rr + 1, 1)
                compute_tail(rr, 0)

                @pl.when(rr + 2 < CPH)
                def _():
                    issue(rr + 2, 0)

                compute_tail(rr + 1, 1)

            pdesc(CPH - 2, 0).wait()
            mdesc(CPH - 2, 0).wait()
            pdesc(CPH - 1, 1).wait()
            mdesc(CPH - 1, 1).wait()

        plsc.subcore_barrier()
        pltpu.sync_copy(
            acc16.at[pl.ds(s * RPS, RPS)],
            ssum_hbm.at[c, pl.ds(s * RPS, RPS)])
        pltpu.sync_copy(
            acc128.at[pl.ds(s * RPS, RPS)],
            out_hbm.at[c, pl.ds(s * RPS, RPS)])

    return k(xn, tgt2d, src2d, ka1, tba, zeros16, zeros128)


def _tc_finish(ssum_p, acc, bias):
    rb = 1000

    def body(s_ref, a_ref, b_ref, o_ref):
        rinv = 1.0 / (s_ref[0, :, :H] + s_ref[1, :, :H] + 1e-7)
        rinvx = jnp.repeat(rinv, U, axis=1)
        o_ref[...] = jax.nn.gelu(
            (a_ref[0] + a_ref[1]) * rinvx + b_ref[...])

    return pl.pallas_call(
        body,
        grid=(N // rb,),
        in_specs=[
            pl.BlockSpec((NC, rb, HP), lambda i: (0, i, 0)),
            pl.BlockSpec((NC, rb, D), lambda i: (0, i, 0)),
            pl.BlockSpec((1, D), lambda i: (0, 0)),
        ],
        out_specs=pl.BlockSpec((rb, D), lambda i: (i, 0)),
        out_shape=jax.ShapeDtypeStruct((N, D), jnp.float32),
    )(ssum_p, acc, bias.reshape(1, D))


def kernel(x, edges, kernel, kernel_attention1, bias_attention, bias):
    w = kernel.reshape(D, H * U)
    ka1 = kernel_attention1.reshape(H * U)
    tba = 2.0 * bias_attention.reshape(H * U)
    pad = N + (jnp.arange(E_PAD - E, dtype=jnp.int32) % (NP - N))
    tgt2d = jnp.concatenate([edges[:, 1], pad]).reshape(NCH, CHUNK)
    src2d = jnp.concatenate([edges[:, 0], pad]).reshape(NCH, CHUNK)
    xp = jnp.pad(x, ((0, NP - N), (0, 0)))
    zeros16 = jnp.zeros((NP, HP), jnp.float32)
    zeros128 = jnp.zeros((NP, D), jnp.float32)

    xn = _tc_project(xp, w)
    ssum_p, acc = _sc_fused(xn, tgt2d, src2d, ka1, tba, zeros16, zeros128)
    return _tc_finish(ssum_p, acc, bias)
